# baseline (speedup 1.0000x reference)
import functools

import jax
import jax.numpy as jnp
from jax import lax
from jax.experimental import pallas as pl
from jax.experimental.pallas import tpu as pltpu
from jax.experimental.pallas import tpu_sc as plsc

NUM_MODELS = 4
VOCAB = 100000
RANK = 8
DIM = 32
BATCH = 1024
POS = 20
SEQ = BATCH * POS
NTOT = NUM_MODELS * SEQ

NC, NS, LANES = 2, 16, 16
NW = NC * NS
B_PER_W = NTOT // NW


V1, V2 = 800, 125
V1_BLK = 40


def _tc_detranspose(At):
    At4 = At.reshape(NUM_MODELS, RANK, V1, V2)

    def body(a_ref, o_ref):
        o_ref[0] = jnp.transpose(a_ref[0], (1, 2, 0))

    t4 = pl.pallas_call(
        body,
        grid=(NUM_MODELS, V1 // V1_BLK),
        in_specs=[pl.BlockSpec((1, RANK, V1_BLK, V2),
                               lambda m, c: (m, 0, c, 0))],
        out_specs=pl.BlockSpec((1, V1_BLK, V2, RANK),
                               lambda m, c: (m, c, 0, 0)),
        out_shape=jax.ShapeDtypeStruct((NUM_MODELS, V1, V2, RANK),
                                       jnp.float32),
    )(At4)
    return t4.reshape(NUM_MODELS, VOCAB, RANK)


CHUNK = 512
N_CHUNKS = B_PER_W // CHUNK
TW = NUM_MODELS * RANK


def _sc_gather(iflat, tab3):
    mesh = plsc.VectorSubcoreMesh(core_axis_name="c", subcore_axis_name="s")

    @functools.partial(
        pl.kernel,
        mesh=mesh,
        compiler_params=pltpu.CompilerParams(use_tc_tiling_on_sc=False),
        out_type=jax.ShapeDtypeStruct(
            (POS, NUM_MODELS, BATCH, NUM_MODELS, RANK), jnp.float32),
        scratch_types=[
            pltpu.VMEM((B_PER_W,), jnp.int32),
            pltpu.VMEM((B_PER_W, NUM_MODELS, RANK), jnp.float32),
            pltpu.SemaphoreType.DMA,
        ],
    )
    def gather_kernel(idx_hbm, tab_hbm, o_hbm, idx_v, rows_v, sem):
        wid = lax.axis_index("s") * NC + lax.axis_index("c")
        base = wid * B_PER_W
        pltpu.sync_copy(idx_hbm.at[pl.ds(base, B_PER_W)], idx_v)
        pltpu.async_copy(tab_hbm.at[idx_v], rows_v, sem).wait()

        for j in range(N_CHUNKS):
            p = base + j * CHUNK
            t = p // (NUM_MODELS * BATCH)
            m = (p // BATCH) % NUM_MODELS
            b0 = p % BATCH
            pltpu.sync_copy(rows_v.at[pl.ds(j * CHUNK, CHUNK)],
                            o_hbm.at[t, m, pl.ds(b0, CHUNK)])

    return gather_kernel(iflat, tab3)


def _tc_matmul(g, Bblk):
    def body(b_ref, g_ref, o_ref):
        bsel = b_ref[...].reshape(NUM_MODELS, TW, DIM)
        gg = g_ref[0].reshape(NUM_MODELS, BATCH, TW)
        o_ref[:, 0] = lax.dot_general(
            bsel, gg, (((1,), (2,)), ((0,), (0,))),
            preferred_element_type=jnp.float32)

    return pl.pallas_call(
        body,
        grid=(POS,),
        in_specs=[
            pl.BlockSpec((NUM_MODELS, NUM_MODELS, RANK, DIM),
                         lambda t: (0, 0, 0, 0)),
            pl.BlockSpec((1, NUM_MODELS, BATCH, NUM_MODELS, RANK),
                         lambda t: (t, 0, 0, 0, 0)),
        ],
        out_specs=pl.BlockSpec((NUM_MODELS, 1, DIM, BATCH),
                               lambda t: (0, t, 0, 0)),
        out_shape=jax.ShapeDtypeStruct((NUM_MODELS, POS, DIM, BATCH),
                                       jnp.float32),
    )(Bblk, g)


def kernel(idx, A, B):
    m, b, t = idx.shape
    iflat = jnp.transpose(idx.astype(jnp.int32), (2, 0, 1)).reshape(-1)
    tab3 = jnp.transpose(A, (1, 0, 2))
    mask = (jnp.arange(NUM_MODELS)[None, :, None, None]
            == jnp.arange(NUM_MODELS)[:, None, None, None])
    Bblk = jnp.where(mask, B[:, None, :, :], 0.0)
    g = _sc_gather(iflat, tab3)
    out_t = _tc_matmul(g, Bblk)
    return jnp.transpose(out_t, (0, 3, 1, 2))


# speedup vs baseline: 3.4907x; 3.4907x over previous
import functools

import jax
import jax.numpy as jnp
from jax import lax
from jax.experimental import pallas as pl
from jax.experimental.pallas import tpu as pltpu
from jax.experimental.pallas import tpu_sc as plsc

NUM_MODELS = 4
VOCAB = 100000
RANK = 8
DIM = 32
BATCH = 1024
POS = 20
SEQ = BATCH * POS
NTOT = NUM_MODELS * SEQ

NC, NS, LANES = 2, 16, 16
NW = NC * NS
B_PER_W = NTOT // NW


V1, V2 = 800, 125
V1_BLK = 40


def _tc_detranspose(At):
    At4 = At.reshape(NUM_MODELS, RANK, V1, V2)

    def body(a_ref, o_ref):
        o_ref[0] = jnp.transpose(a_ref[0], (1, 2, 0))

    t4 = pl.pallas_call(
        body,
        grid=(NUM_MODELS, V1 // V1_BLK),
        in_specs=[pl.BlockSpec((1, RANK, V1_BLK, V2),
                               lambda m, c: (m, 0, c, 0))],
        out_specs=pl.BlockSpec((1, V1_BLK, V2, RANK),
                               lambda m, c: (m, c, 0, 0)),
        out_shape=jax.ShapeDtypeStruct((NUM_MODELS, V1, V2, RANK),
                                       jnp.float32),
    )(At4)
    return t4.reshape(NUM_MODELS, VOCAB, RANK)


CHUNK = 512
N_CHUNKS = B_PER_W // CHUNK
TW = NUM_MODELS * RANK


def _sc_gather(iflat, table32):
    mesh = plsc.VectorSubcoreMesh(core_axis_name="c", subcore_axis_name="s")

    @functools.partial(
        pl.kernel,
        mesh=mesh,
        compiler_params=pltpu.CompilerParams(use_tc_tiling_on_sc=False),
        out_type=jax.ShapeDtypeStruct((POS, NUM_MODELS, BATCH, TW),
                                      jnp.bfloat16),
        scratch_types=[
            pltpu.VMEM((B_PER_W,), jnp.int32),
            pltpu.VMEM((B_PER_W, TW), jnp.bfloat16),
            pltpu.SemaphoreType.DMA,
        ],
    )
    def gather_kernel(idx_hbm, tab_hbm, o_hbm, idx_v, rows_v, sem):
        wid = lax.axis_index("s") * NC + lax.axis_index("c")
        base = wid * B_PER_W
        pltpu.sync_copy(idx_hbm.at[pl.ds(base, B_PER_W)], idx_v)
        pltpu.async_copy(tab_hbm.at[idx_v], rows_v, sem).wait()

        for j in range(N_CHUNKS):
            p = base + j * CHUNK
            t = p // (NUM_MODELS * BATCH)
            m = (p // BATCH) % NUM_MODELS
            b0 = p % BATCH
            pltpu.sync_copy(rows_v.at[pl.ds(j * CHUNK, CHUNK)],
                            o_hbm.at[t, m, pl.ds(b0, CHUNK)])

    return gather_kernel(iflat, table32)


def _tc_matmul(g, Bblk):
    def body(b_ref, g_ref, o_ref):
        o_ref[:, 0] = lax.dot_general(
            b_ref[...], g_ref[0], (((1,), (2,)), ((0,), (0,))),
            preferred_element_type=jnp.float32)

    return pl.pallas_call(
        body,
        grid=(POS,),
        in_specs=[
            pl.BlockSpec((NUM_MODELS, TW, DIM), lambda t: (0, 0, 0)),
            pl.BlockSpec((1, NUM_MODELS, BATCH, TW), lambda t: (t, 0, 0, 0)),
        ],
        out_specs=pl.BlockSpec((NUM_MODELS, 1, DIM, BATCH),
                               lambda t: (0, t, 0, 0)),
        out_shape=jax.ShapeDtypeStruct((NUM_MODELS, POS, DIM, BATCH),
                                       jnp.float32),
    )(Bblk, g)


def kernel(idx, A, B):
    m, b, t = idx.shape
    iflat = jnp.transpose(idx.astype(jnp.int32), (2, 0, 1)).reshape(-1)
    table32 = (jnp.transpose(A.astype(jnp.bfloat16), (1, 0, 2))
               .reshape(VOCAB, TW))
    mask = ((jnp.arange(TW) // RANK)[None, :, None]
            == jnp.arange(NUM_MODELS)[:, None, None])
    Bblk = jnp.where(mask, jnp.tile(B, (1, NUM_MODELS, 1)),
                     0.0).astype(jnp.bfloat16)
    g = _sc_gather(iflat, table32)
    out_t = _tc_matmul(g, Bblk)
    return jnp.transpose(out_t, (0, 3, 1, 2))


# speedup vs baseline: 4.1450x; 1.1874x over previous
import functools

import jax
import jax.numpy as jnp
from jax import lax
from jax.experimental import pallas as pl
from jax.experimental.pallas import tpu as pltpu
from jax.experimental.pallas import tpu_sc as plsc

NUM_MODELS = 4
VOCAB = 100000
RANK = 8
DIM = 32
BATCH = 1024
POS = 20
SEQ = BATCH * POS
NTOT = NUM_MODELS * SEQ

NC, NS, LANES = 2, 16, 16
NW = NC * NS
B_PER_W = NTOT // NW


V1, V2 = 800, 125
V1_BLK = 40


def _tc_detranspose(At):
    At4 = At.reshape(NUM_MODELS, RANK, V1, V2)

    def body(a_ref, o_ref):
        o_ref[0] = jnp.transpose(a_ref[0], (1, 2, 0))

    t4 = pl.pallas_call(
        body,
        grid=(NUM_MODELS, V1 // V1_BLK),
        in_specs=[pl.BlockSpec((1, RANK, V1_BLK, V2),
                               lambda m, c: (m, 0, c, 0))],
        out_specs=pl.BlockSpec((1, V1_BLK, V2, RANK),
                               lambda m, c: (m, c, 0, 0)),
        out_shape=jax.ShapeDtypeStruct((NUM_MODELS, V1, V2, RANK),
                                       jnp.float32),
    )(At4)
    return t4.reshape(NUM_MODELS, VOCAB, RANK)


CHUNK = 512
N_CHUNKS = B_PER_W // CHUNK
TW = NUM_MODELS * RANK


def _sc_gather(iflat, table32):
    mesh = plsc.VectorSubcoreMesh(core_axis_name="c", subcore_axis_name="s")

    @functools.partial(
        pl.kernel,
        mesh=mesh,
        compiler_params=pltpu.CompilerParams(use_tc_tiling_on_sc=False),
        out_type=jax.ShapeDtypeStruct((POS, NUM_MODELS, BATCH, TW),
                                      jnp.float32),
        scratch_types=[
            pltpu.VMEM((B_PER_W,), jnp.int32),
            pltpu.VMEM((B_PER_W, TW), jnp.float32),
            pltpu.SemaphoreType.DMA,
        ],
    )
    def gather_kernel(idx_hbm, tab_hbm, o_hbm, idx_v, rows_v, sem):
        wid = lax.axis_index("s") * NC + lax.axis_index("c")
        base = wid * B_PER_W
        pltpu.sync_copy(idx_hbm.at[pl.ds(base, B_PER_W)], idx_v)
        pltpu.async_copy(tab_hbm.at[idx_v], rows_v, sem).wait()

        for j in range(N_CHUNKS):
            p = base + j * CHUNK
            t = p // (NUM_MODELS * BATCH)
            m = (p // BATCH) % NUM_MODELS
            b0 = p % BATCH
            pltpu.sync_copy(rows_v.at[pl.ds(j * CHUNK, CHUNK)],
                            o_hbm.at[t, m, pl.ds(b0, CHUNK)])

    return gather_kernel(iflat, table32)


def _tc_matmul(g, Bblk):
    def body(b_ref, g_ref, o_ref):
        o_ref[:, 0] = lax.dot_general(
            b_ref[...].astype(jnp.bfloat16), g_ref[0].astype(jnp.bfloat16),
            (((1,), (2,)), ((0,), (0,))),
            preferred_element_type=jnp.float32)

    return pl.pallas_call(
        body,
        grid=(POS,),
        in_specs=[
            pl.BlockSpec((NUM_MODELS, TW, DIM), lambda t: (0, 0, 0)),
            pl.BlockSpec((1, NUM_MODELS, BATCH, TW), lambda t: (t, 0, 0, 0)),
        ],
        out_specs=pl.BlockSpec((NUM_MODELS, 1, DIM, BATCH),
                               lambda t: (0, t, 0, 0)),
        out_shape=jax.ShapeDtypeStruct((NUM_MODELS, POS, DIM, BATCH),
                                       jnp.float32),
    )(Bblk, g)


def kernel(idx, A, B):
    m, b, t = idx.shape
    iflat = jnp.transpose(idx.astype(jnp.int32), (2, 0, 1)).reshape(-1)
    table32 = jnp.transpose(A, (1, 0, 2)).reshape(VOCAB, TW)
    mask = ((jnp.arange(TW) // RANK)[None, :, None]
            == jnp.arange(NUM_MODELS)[:, None, None])
    Bblk = jnp.where(mask, jnp.tile(B, (1, NUM_MODELS, 1)), 0.0)
    g = _sc_gather(iflat, table32)
    out_t = _tc_matmul(g, Bblk)
    return jnp.transpose(out_t, (0, 3, 1, 2))


# speedup vs baseline: 4.1524x; 1.0018x over previous
import functools

import jax
import jax.numpy as jnp
from jax import lax
from jax.experimental import pallas as pl
from jax.experimental.pallas import tpu as pltpu
from jax.experimental.pallas import tpu_sc as plsc

NUM_MODELS = 4
VOCAB = 100000
RANK = 8
DIM = 32
BATCH = 1024
POS = 20
SEQ = BATCH * POS
NTOT = NUM_MODELS * SEQ

NC, NS, LANES = 2, 16, 16
NW = NC * NS
B_PER_W = NTOT // NW


V1, V2 = 800, 125
V1_BLK = 40


def _tc_detranspose(At):
    At4 = At.reshape(NUM_MODELS, RANK, V1, V2)

    def body(a_ref, o_ref):
        o_ref[0] = jnp.transpose(a_ref[0], (1, 2, 0))

    t4 = pl.pallas_call(
        body,
        grid=(NUM_MODELS, V1 // V1_BLK),
        in_specs=[pl.BlockSpec((1, RANK, V1_BLK, V2),
                               lambda m, c: (m, 0, c, 0))],
        out_specs=pl.BlockSpec((1, V1_BLK, V2, RANK),
                               lambda m, c: (m, c, 0, 0)),
        out_shape=jax.ShapeDtypeStruct((NUM_MODELS, V1, V2, RANK),
                                       jnp.float32),
    )(At4)
    return t4.reshape(NUM_MODELS, VOCAB, RANK)


CHUNK = 512
N_CHUNKS = B_PER_W // CHUNK
TW = NUM_MODELS * RANK


def _sc_gather(iflat, table32):
    mesh = plsc.VectorSubcoreMesh(core_axis_name="c", subcore_axis_name="s")

    @functools.partial(
        pl.kernel,
        mesh=mesh,
        compiler_params=pltpu.CompilerParams(use_tc_tiling_on_sc=False),
        out_type=jax.ShapeDtypeStruct((POS, NUM_MODELS, BATCH, TW),
                                      jnp.float32),
        scratch_types=[
            pltpu.VMEM((B_PER_W,), jnp.int32),
            pltpu.VMEM((B_PER_W, TW), jnp.float32),
            pltpu.SemaphoreType.DMA,
        ],
    )
    def gather_kernel(idx_hbm, tab_hbm, o_hbm, idx_v, rows_v, sem):
        wid = lax.axis_index("s") * NC + lax.axis_index("c")
        base = wid * B_PER_W
        pltpu.sync_copy(idx_hbm.at[pl.ds(base, B_PER_W)], idx_v)
        pltpu.async_copy(tab_hbm.at[idx_v], rows_v, sem).wait()

        for j in range(N_CHUNKS):
            p = base + j * CHUNK
            t = p // (NUM_MODELS * BATCH)
            m = (p // BATCH) % NUM_MODELS
            b0 = p % BATCH
            pltpu.sync_copy(rows_v.at[pl.ds(j * CHUNK, CHUNK)],
                            o_hbm.at[t, m, pl.ds(b0, CHUNK)])

    return gather_kernel(iflat, table32)


def _tc_matmul(g, Bblk):
    def body(b_ref, g_ref, o_ref):
        o_ref[:, 0] = lax.dot_general(
            b_ref[...], g_ref[0], (((1,), (2,)), ((0,), (0,))),
            preferred_element_type=jnp.float32)

    return pl.pallas_call(
        body,
        grid=(POS,),
        in_specs=[
            pl.BlockSpec((NUM_MODELS, TW, DIM), lambda t: (0, 0, 0)),
            pl.BlockSpec((1, NUM_MODELS, BATCH, TW), lambda t: (t, 0, 0, 0)),
        ],
        out_specs=pl.BlockSpec((NUM_MODELS, 1, DIM, BATCH),
                               lambda t: (0, t, 0, 0)),
        out_shape=jax.ShapeDtypeStruct((NUM_MODELS, POS, DIM, BATCH),
                                       jnp.float32),
    )(Bblk, g)


def kernel(idx, A, B):
    m, b, t = idx.shape
    iflat = jnp.transpose(idx.astype(jnp.int32), (2, 0, 1)).reshape(-1)
    table32 = jnp.transpose(A, (1, 0, 2)).reshape(VOCAB, TW)
    mask = ((jnp.arange(TW) // RANK)[None, :, None]
            == jnp.arange(NUM_MODELS)[:, None, None])
    Bblk = jnp.where(mask, jnp.tile(B, (1, NUM_MODELS, 1)), 0.0)
    g = _sc_gather(iflat, table32)
    out_t = _tc_matmul(g, Bblk)
    return jnp.transpose(out_t, (0, 3, 1, 2))


# speedup vs baseline: 4.3468x; 1.0468x over previous
import functools

import jax
import jax.numpy as jnp
from jax import lax
from jax.experimental import pallas as pl
from jax.experimental.pallas import tpu as pltpu
from jax.experimental.pallas import tpu_sc as plsc

NUM_MODELS = 4
VOCAB = 100000
RANK = 8
DIM = 32
BATCH = 1024
POS = 20
SEQ = BATCH * POS
NTOT = NUM_MODELS * SEQ

NC, NS, LANES = 2, 16, 16
NW = NC * NS
B_PER_W = NTOT // NW


V1, V2 = 800, 125
V1_BLK = 40


def _tc_detranspose(At):
    At4 = At.reshape(NUM_MODELS, RANK, V1, V2)

    def body(a_ref, o_ref):
        o_ref[0] = jnp.transpose(a_ref[0], (1, 2, 0))

    t4 = pl.pallas_call(
        body,
        grid=(NUM_MODELS, V1 // V1_BLK),
        in_specs=[pl.BlockSpec((1, RANK, V1_BLK, V2),
                               lambda m, c: (m, 0, c, 0))],
        out_specs=pl.BlockSpec((1, V1_BLK, V2, RANK),
                               lambda m, c: (m, c, 0, 0)),
        out_shape=jax.ShapeDtypeStruct((NUM_MODELS, V1, V2, RANK),
                                       jnp.float32),
    )(At4)
    return t4.reshape(NUM_MODELS, VOCAB, RANK)


CHUNK = 512
N_CHUNKS = B_PER_W // CHUNK
TW = NUM_MODELS * RANK


def _sc_gather(iflat, table32):
    mesh = plsc.VectorSubcoreMesh(core_axis_name="c", subcore_axis_name="s")

    @functools.partial(
        pl.kernel,
        mesh=mesh,
        compiler_params=pltpu.CompilerParams(use_tc_tiling_on_sc=False),
        out_type=jax.ShapeDtypeStruct((POS, NUM_MODELS, BATCH, TW),
                                      jnp.float32),
        scratch_types=[
            pltpu.VMEM((B_PER_W,), jnp.int32),
            pltpu.VMEM((B_PER_W, TW), jnp.float32),
            pltpu.SemaphoreType.DMA,
        ],
    )
    def gather_kernel(idx_hbm, tab_hbm, o_hbm, idx_v, rows_v, sem):
        wid = lax.axis_index("s") * NC + lax.axis_index("c")
        base = wid * B_PER_W
        pltpu.sync_copy(idx_hbm.at[pl.ds(base, B_PER_W)], idx_v)
        pltpu.async_copy(tab_hbm.at[idx_v], rows_v, sem).wait()

        for j in range(N_CHUNKS):
            p = base + j * CHUNK
            t = p // (NUM_MODELS * BATCH)
            m = (p // BATCH) % NUM_MODELS
            b0 = p % BATCH
            pltpu.sync_copy(rows_v.at[pl.ds(j * CHUNK, CHUNK)],
                            o_hbm.at[t, m, pl.ds(b0, CHUNK)])

    return gather_kernel(iflat, table32)


def _tc_matmul(g, Bblk):
    TB = 2

    def body(b_ref, g_ref, o_ref):
        for j in range(TB):
            o_ref[:, j] = lax.dot_general(
                b_ref[...], g_ref[j], (((1,), (2,)), ((0,), (0,))),
                preferred_element_type=jnp.float32)

    return pl.pallas_call(
        body,
        grid=(POS // TB,),
        in_specs=[
            pl.BlockSpec((NUM_MODELS, TW, DIM), lambda t: (0, 0, 0)),
            pl.BlockSpec((TB, NUM_MODELS, BATCH, TW), lambda t: (t, 0, 0, 0)),
        ],
        out_specs=pl.BlockSpec((NUM_MODELS, TB, DIM, BATCH),
                               lambda t: (0, t, 0, 0)),
        out_shape=jax.ShapeDtypeStruct((NUM_MODELS, POS, DIM, BATCH),
                                       jnp.float32),
    )(Bblk, g)


def kernel(idx, A, B):
    m, b, t = idx.shape
    iflat = jnp.transpose(idx.astype(jnp.int32), (2, 0, 1)).reshape(-1)
    table32 = jnp.transpose(A, (1, 0, 2)).reshape(VOCAB, TW)
    mask = ((jnp.arange(TW) // RANK)[None, :, None]
            == jnp.arange(NUM_MODELS)[:, None, None])
    Bblk = jnp.where(mask, jnp.tile(B, (1, NUM_MODELS, 1)), 0.0)
    g = _sc_gather(iflat, table32)
    out_t = _tc_matmul(g, Bblk)
    return jnp.transpose(out_t, (0, 3, 1, 2))
